# initial kernel scaffold (unmeasured)
import jax
import jax.numpy as jnp
from jax import lax
from jax.experimental import pallas as pl
from jax.experimental.pallas import tpu as pltpu


def kernel(
    x,
):
    def body(*refs):
        pass

    out_shape = jax.ShapeDtypeStruct(..., jnp.float32)
    return pl.pallas_call(body, out_shape=out_shape)(...)


# baseline (device time: 217293 ns/iter reference)
import jax
import jax.numpy as jnp
from jax import lax
from jax.experimental import pallas as pl
from jax.experimental.pallas import tpu as pltpu


def kernel(x):
    m_per, n = x.shape
    xb = x.astype(jnp.bfloat16)

    def body(x_ref, out_ref, local_sem, send_sem, recv_sem):
        my_x = lax.axis_index("x")
        my_y = lax.axis_index("y")
        other_x = 1 - my_x

        barrier_sem = pltpu.get_barrier_semaphore()
        pl.semaphore_signal(
            barrier_sem,
            inc=1,
            device_id=(other_x, my_y),
            device_id_type=pl.DeviceIdType.MESH,
        )
        pl.semaphore_wait(barrier_sem, 1)

        local = pltpu.make_async_copy(
            x_ref, out_ref.at[pl.ds(my_x * m_per, m_per), :], local_sem
        )
        local.start()

        rdma = pltpu.make_async_remote_copy(
            src_ref=x_ref,
            dst_ref=out_ref.at[pl.ds(my_x * m_per, m_per), :],
            send_sem=send_sem,
            recv_sem=recv_sem,
            device_id=(other_x, my_y),
            device_id_type=pl.DeviceIdType.MESH,
        )
        rdma.start()
        rdma.wait()
        local.wait()

    return pl.pallas_call(
        body,
        out_shape=jax.ShapeDtypeStruct((2 * m_per, n), jnp.bfloat16),
        in_specs=[pl.BlockSpec(memory_space=pl.ANY)],
        out_specs=pl.BlockSpec(memory_space=pl.ANY),
        scratch_shapes=[
            pltpu.SemaphoreType.DMA,
            pltpu.SemaphoreType.DMA,
            pltpu.SemaphoreType.DMA,
        ],
        compiler_params=pltpu.CompilerParams(collective_id=0),
    )(xb)


# device time: 136580 ns/iter; 1.5910x vs baseline; 1.5910x over previous
import jax
import jax.numpy as jnp
from jax import lax
from jax.experimental import pallas as pl
from jax.experimental.pallas import tpu as pltpu

N_CHUNKS = 16


def kernel(x):
    m_per, n = x.shape
    half = m_per // 2
    ch = half // N_CHUNKS
    xb = x.astype(jnp.bfloat16)

    def body(x_ref, out_ref, local_sem, xs_sems, xr_sems, ys_sems, yr_sems):
        my_x = lax.axis_index("x")
        my_y = lax.axis_index("y")
        x_nbr = (1 - my_x, my_y)
        y_nbr = (my_x, 1 - my_y)
        out_me = my_x * m_per
        out_other = (1 - my_x) * m_per
        h0 = my_y * half
        oh0 = (1 - my_y) * half

        barrier_sem = pltpu.get_barrier_semaphore()
        for nbr in (x_nbr, y_nbr):
            pl.semaphore_signal(
                barrier_sem, inc=1, device_id=nbr,
                device_id_type=pl.DeviceIdType.MESH,
            )
        pl.semaphore_wait(barrier_sem, 2)

        x_sends = []
        for c in range(N_CHUNKS):
            r = h0 + c * ch
            s = pltpu.make_async_remote_copy(
                src_ref=x_ref.at[pl.ds(r, ch), :],
                dst_ref=out_ref.at[pl.ds(out_me + r, ch), :],
                send_sem=xs_sems.at[c],
                recv_sem=xr_sems.at[c],
                device_id=x_nbr,
                device_id_type=pl.DeviceIdType.MESH,
            )
            s.start()
            x_sends.append(s)

        local = pltpu.make_async_copy(
            x_ref, out_ref.at[pl.ds(out_me, m_per), :], local_sem
        )
        local.start()

        y_sends = []
        for c in range(N_CHUNKS):
            r = out_other + h0 + c * ch
            recv = pltpu.make_async_remote_copy(
                src_ref=out_ref.at[pl.ds(r, ch), :],
                dst_ref=out_ref.at[pl.ds(r, ch), :],
                send_sem=ys_sems.at[c],
                recv_sem=xr_sems.at[c],
                device_id=x_nbr,
                device_id_type=pl.DeviceIdType.MESH,
            )
            recv.wait_recv()
            fwd = pltpu.make_async_remote_copy(
                src_ref=out_ref.at[pl.ds(r, ch), :],
                dst_ref=out_ref.at[pl.ds(r, ch), :],
                send_sem=ys_sems.at[c],
                recv_sem=yr_sems.at[c],
                device_id=y_nbr,
                device_id_type=pl.DeviceIdType.MESH,
            )
            fwd.start()
            y_sends.append(fwd)

        for c in range(N_CHUNKS):
            r = out_other + oh0 + c * ch
            yrecv = pltpu.make_async_remote_copy(
                src_ref=out_ref.at[pl.ds(r, ch), :],
                dst_ref=out_ref.at[pl.ds(r, ch), :],
                send_sem=ys_sems.at[c],
                recv_sem=yr_sems.at[c],
                device_id=y_nbr,
                device_id_type=pl.DeviceIdType.MESH,
            )
            yrecv.wait_recv()

        for s in x_sends:
            s.wait_send()
        for s in y_sends:
            s.wait_send()
        local.wait()

    return pl.pallas_call(
        body,
        out_shape=jax.ShapeDtypeStruct((2 * m_per, n), jnp.bfloat16),
        in_specs=[pl.BlockSpec(memory_space=pl.ANY)],
        out_specs=pl.BlockSpec(memory_space=pl.ANY),
        scratch_shapes=[
            pltpu.SemaphoreType.DMA,
            pltpu.SemaphoreType.DMA((N_CHUNKS,)),
            pltpu.SemaphoreType.DMA((N_CHUNKS,)),
            pltpu.SemaphoreType.DMA((N_CHUNKS,)),
            pltpu.SemaphoreType.DMA((N_CHUNKS,)),
        ],
        compiler_params=pltpu.CompilerParams(collective_id=0),
    )(xb)
